# baseline (speedup 1.0000x reference)
import functools
import jax
import jax.numpy as jnp
from jax import lax
from jax.experimental import pallas as pl
from jax.experimental.pallas import tpu as pltpu
from jax.experimental.pallas import tpu_sc as plsc

_N = 8192
_NUMEL = _N * _N
_NW = 32
_E = _NUMEL // _NW
_CHUNK = 16384
_NCHUNKS = _E // _CHUNK
_VECS = _CHUNK // 16
_UNROLL = 16

_mesh = plsc.VectorSubcoreMesh(core_axis_name="c", subcore_axis_name="s")


@functools.partial(
    pl.kernel,
    mesh=_mesh,
    out_type=jax.ShapeDtypeStruct((_NW, 16), jnp.float32),
    scratch_types=[
        pltpu.VMEM((_CHUNK,), jnp.float32),
        pltpu.VMEM((_CHUNK,), jnp.float32),
        pltpu.VMEM((_CHUNK,), jnp.float32),
        pltpu.VMEM((_CHUNK,), jnp.float32),
        pltpu.VMEM((16,), jnp.float32),
        pltpu.SemaphoreType.DMA,
        pltpu.SemaphoreType.DMA,
    ],
)
def _sc_body(pred, target, out, pb0, tb0, pb1, tb1, accbuf, sem0, sem1):
    c = lax.axis_index("c")
    s = lax.axis_index("s")
    wid = s * 2 + c
    base = wid * _E

    def start(buf_p, buf_t, sem, off):
        pltpu.async_copy(pred.at[pl.ds(off, _CHUNK)], buf_p, sem)
        pltpu.async_copy(target.at[pl.ds(off, _CHUNK)], buf_t, sem)

    def wait(buf_p, buf_t, sem):
        pltpu.make_async_copy(pred.at[pl.ds(0, _CHUNK)], buf_p, sem).wait()
        pltpu.make_async_copy(target.at[pl.ds(0, _CHUNK)], buf_t, sem).wait()

    def consume(buf_p, buf_t, accs):
        @plsc.parallel_loop(0, _VECS, step=_UNROLL, unroll=2, carry=accs)
        def inner(i, acc_t):
            a0, a1, a2, a3 = acc_t
            acc4 = [a0, a1, a2, a3]
            for u in range(_UNROLL):
                idx = (i + u) * 16
                p = buf_p[pl.ds(idx, 16)]
                t = buf_t[pl.ds(idx, 16)]
                d = jnp.minimum(jnp.maximum(p, 0.0), 1.0) - t
                acc4[u % 4] = acc4[u % 4] + d * d
            return tuple(acc4)

        return inner

    start(pb0, tb0, sem0, base)

    zero = jnp.zeros((16,), jnp.float32)

    def body(g, accs):
        start(pb1, tb1, sem1, base + (2 * g + 1) * _CHUNK)
        wait(pb0, tb0, sem0)

        @pl.when(g < _NCHUNKS // 2 - 1)
        def _():
            start(pb0, tb0, sem0, base + (2 * g + 2) * _CHUNK)

        wait(pb1, tb1, sem1)
        return accs

    accs = lax.fori_loop(0, _NCHUNKS // 2, body, (zero, zero, zero, zero))
    accbuf[...] = (accs[0] + accs[1]) + (accs[2] + accs[3])
    pltpu.sync_copy(accbuf, out.at[wid])


def kernel(pred, target):
    out = _sc_body(pred.reshape(-1), target.reshape(-1))
    return jnp.sum(out) * (1.0 / float(_NUMEL))


# speedup vs baseline: 2.6145x; 2.6145x over previous
import jax
import jax.numpy as jnp
from jax import lax
from jax.experimental import pallas as pl
from jax.experimental.pallas import tpu as pltpu
from jax.experimental.pallas import tpu_sc as plsc

_N = 8192
_NUMEL = _N * _N
_NW = 32
_ROWS_PER_W = _N // _NW
_CR = 8
_CC = 2048
_CHUNK = _CR * _CC
_NCHUNKS = _ROWS_PER_W * _N // _CHUNK
_QPS = _N // _CC
_VECS_PER_ROW = _CC // 16

_mesh = plsc.VectorSubcoreMesh(core_axis_name="c", subcore_axis_name="s")


def _sc_partial(pred, target):
    @pl.kernel(
        out_type=jax.ShapeDtypeStruct((_NW, 16), jnp.float32),
        mesh=_mesh,
        scratch_types=[
            pltpu.VMEM((_CR, _CC), jnp.float32),
            pltpu.VMEM((_CR, _CC), jnp.float32),
            pltpu.VMEM((_CR, _CC), jnp.float32),
            pltpu.VMEM((_CR, _CC), jnp.float32),
            pltpu.VMEM((16,), jnp.float32),
            pltpu.SemaphoreType.DMA,
            pltpu.SemaphoreType.DMA,
        ],
        compiler_params=pltpu.CompilerParams(use_tc_tiling_on_sc=True),
    )
    def body(pred, target, out, pb0, tb0, pb1, tb1, accbuf, sem0, sem1):
        c = lax.axis_index("c")
        s = lax.axis_index("s")
        wid = s * 2 + c
        row0 = wid * _ROWS_PER_W

        def start(buf_p, buf_t, sem, ci):
            row = row0 + (ci >> 2) * _CR
            col = (ci & 3) * _CC
            pltpu.async_copy(
                pred.at[pl.ds(row, _CR), pl.ds(col, _CC)], buf_p, sem
            )
            pltpu.async_copy(
                target.at[pl.ds(row, _CR), pl.ds(col, _CC)], buf_t, sem
            )

        def wait(buf_p, buf_t, sem):
            pltpu.make_async_copy(
                pred.at[pl.ds(0, _CR), pl.ds(0, _CC)], buf_p, sem
            ).wait()
            pltpu.make_async_copy(
                target.at[pl.ds(0, _CR), pl.ds(0, _CC)], buf_t, sem
            ).wait()

        def consume(buf_p, buf_t, accs):
            for r in range(_CR):

                @plsc.parallel_loop(
                    0, _VECS_PER_ROW, step=16, unroll=2, carry=accs
                )
                def inner(i, acc_t):
                    acc4 = list(acc_t)
                    for u in range(16):
                        idx = (i + u) * 16
                        p = buf_p[r, pl.ds(idx, 16)]
                        t = buf_t[r, pl.ds(idx, 16)]
                        d = jnp.minimum(jnp.maximum(p, 0.0), 1.0) - t
                        acc4[u % 4] = acc4[u % 4] + d * d
                    return tuple(acc4)

                accs = inner
            return accs

        start(pb0, tb0, sem0, 0)

        zero = jnp.zeros((16,), jnp.float32)

        def pair(g, accs):
            start(pb1, tb1, sem1, 2 * g + 1)
            wait(pb0, tb0, sem0)
            accs = consume(pb0, tb0, accs)

            @pl.when(g < _NCHUNKS // 2 - 1)
            def _():
                start(pb0, tb0, sem0, 2 * g + 2)

            wait(pb1, tb1, sem1)
            accs = consume(pb1, tb1, accs)
            return accs

        accs = lax.fori_loop(0, _NCHUNKS // 2, pair, (zero, zero, zero, zero))
        accbuf[...] = (accs[0] + accs[1]) + (accs[2] + accs[3])
        pltpu.sync_copy(accbuf, out.at[wid])

    return body(pred, target)


def kernel(pred, target):
    out = _sc_partial(pred, target)
    return jnp.sum(out) * (1.0 / float(_NUMEL))


# speedup vs baseline: 3.3434x; 1.2788x over previous
import jax
import jax.numpy as jnp
from jax import lax
from jax.experimental import pallas as pl
from jax.experimental.pallas import tpu as pltpu
from jax.experimental.pallas import tpu_sc as plsc

_N = 8192
_NUMEL = _N * _N

_SC_SLABS = 26
_BLOCK_ROWS = 128
_TC_GRID = (_N // _BLOCK_ROWS) - _SC_SLABS
_TC_ROWS = _TC_GRID * _BLOCK_ROWS

_NW = 32
_ROWS_PER_W = (_N - _TC_ROWS) // _NW
_CR = 8
_CC = 2048
_CHUNK = _CR * _CC
_NCHUNKS = _ROWS_PER_W * _N // _CHUNK
_VECS_PER_ROW = _CC // 16

assert _ROWS_PER_W % _CR == 0 and _NCHUNKS % 2 == 0

_mesh = plsc.VectorSubcoreMesh(core_axis_name="c", subcore_axis_name="s")


def _sc_partial(pred, target):
    @pl.kernel(
        out_type=jax.ShapeDtypeStruct((_NW, 16), jnp.float32),
        mesh=_mesh,
        scratch_types=[
            pltpu.VMEM((_CR, _CC), jnp.float32),
            pltpu.VMEM((_CR, _CC), jnp.float32),
            pltpu.VMEM((_CR, _CC), jnp.float32),
            pltpu.VMEM((_CR, _CC), jnp.float32),
            pltpu.VMEM((16,), jnp.float32),
            pltpu.SemaphoreType.DMA,
            pltpu.SemaphoreType.DMA,
        ],
        compiler_params=pltpu.CompilerParams(use_tc_tiling_on_sc=True),
    )
    def body(pred, target, out, pb0, tb0, pb1, tb1, accbuf, sem0, sem1):
        c = lax.axis_index("c")
        s = lax.axis_index("s")
        wid = s * 2 + c
        row0 = _TC_ROWS + wid * _ROWS_PER_W

        def start(buf_p, buf_t, sem, ci):
            row = row0 + (ci >> 2) * _CR
            col = (ci & 3) * _CC
            pltpu.async_copy(
                pred.at[pl.ds(row, _CR), pl.ds(col, _CC)], buf_p, sem
            )
            pltpu.async_copy(
                target.at[pl.ds(row, _CR), pl.ds(col, _CC)], buf_t, sem
            )

        def wait(buf_p, buf_t, sem):
            pltpu.make_async_copy(
                pred.at[pl.ds(0, _CR), pl.ds(0, _CC)], buf_p, sem
            ).wait()
            pltpu.make_async_copy(
                target.at[pl.ds(0, _CR), pl.ds(0, _CC)], buf_t, sem
            ).wait()

        def consume(buf_p, buf_t, accs):
            for r in range(_CR):

                @plsc.parallel_loop(
                    0, _VECS_PER_ROW, step=16, unroll=2, carry=accs
                )
                def inner(i, acc_t):
                    acc4 = list(acc_t)
                    for u in range(16):
                        idx = (i + u) * 16
                        p = buf_p[r, pl.ds(idx, 16)]
                        t = buf_t[r, pl.ds(idx, 16)]
                        d = jnp.minimum(jnp.maximum(p, 0.0), 1.0) - t
                        acc4[u % 4] = acc4[u % 4] + d * d
                    return tuple(acc4)

                accs = inner
            return accs

        start(pb0, tb0, sem0, 0)

        zero = jnp.zeros((16,), jnp.float32)

        def pair(g, accs):
            start(pb1, tb1, sem1, 2 * g + 1)
            wait(pb0, tb0, sem0)
            accs = consume(pb0, tb0, accs)

            @pl.when(g < _NCHUNKS // 2 - 1)
            def _():
                start(pb0, tb0, sem0, 2 * g + 2)

            wait(pb1, tb1, sem1)
            accs = consume(pb1, tb1, accs)
            return accs

        accs = lax.fori_loop(0, _NCHUNKS // 2, pair, (zero, zero, zero, zero))
        accbuf[...] = (accs[0] + accs[1]) + (accs[2] + accs[3])
        pltpu.sync_copy(accbuf, out.at[wid])

    return body(pred, target)


def _tc_body(p_ref, t_ref, o_ref, acc_ref):
    i = pl.program_id(0)

    @pl.when(i == 0)
    def _init():
        acc_ref[...] = jnp.zeros_like(acc_ref)

    d = jnp.clip(p_ref[...], 0.0, 1.0) - t_ref[...]
    acc_ref[...] += jnp.sum(d * d, axis=0, keepdims=True)

    @pl.when(i == _TC_GRID - 1)
    def _fin():
        o_ref[0, 0] = jnp.sum(acc_ref[...])


def _tc_partial(pred, target):
    return pl.pallas_call(
        _tc_body,
        grid=(_TC_GRID,),
        in_specs=[
            pl.BlockSpec((_BLOCK_ROWS, _N), lambda i: (i, 0)),
            pl.BlockSpec((_BLOCK_ROWS, _N), lambda i: (i, 0)),
        ],
        out_specs=pl.BlockSpec(memory_space=pltpu.SMEM),
        out_shape=jax.ShapeDtypeStruct((1, 1), jnp.float32),
        scratch_shapes=[pltpu.VMEM((1, _N), jnp.float32)],
        compiler_params=pltpu.CompilerParams(
            dimension_semantics=("arbitrary",),
        ),
    )(pred, target)


def kernel(pred, target):
    sc_out = _sc_partial(pred, target)
    tc_out = _tc_partial(pred, target)
    return (tc_out[0, 0] + jnp.sum(sc_out)) * (1.0 / float(_NUMEL))


# speedup vs baseline: 3.3916x; 1.0144x over previous
import jax
import jax.numpy as jnp
from jax import lax
from jax.experimental import pallas as pl
from jax.experimental.pallas import tpu as pltpu
from jax.experimental.pallas import tpu_sc as plsc

_N = 8192
_NUMEL = _N * _N

_SC_SLABS = 16
_BLOCK_ROWS = 128
_TC_GRID = (_N // _BLOCK_ROWS) - _SC_SLABS
_TC_ROWS = _TC_GRID * _BLOCK_ROWS

_NW = 32
_ROWS_PER_W = (_N - _TC_ROWS) // _NW
_CR = 8
_CC = 2048
_CHUNK = _CR * _CC
_NCHUNKS = _ROWS_PER_W * _N // _CHUNK
_VECS_PER_ROW = _CC // 16

assert _ROWS_PER_W % _CR == 0 and _NCHUNKS % 2 == 0

_mesh = plsc.VectorSubcoreMesh(core_axis_name="c", subcore_axis_name="s")


def _sc_partial(pred, target):
    @pl.kernel(
        out_type=jax.ShapeDtypeStruct((_NW, 16), jnp.float32),
        mesh=_mesh,
        scratch_types=[
            pltpu.VMEM((_CR, _CC), jnp.float32),
            pltpu.VMEM((_CR, _CC), jnp.float32),
            pltpu.VMEM((_CR, _CC), jnp.float32),
            pltpu.VMEM((_CR, _CC), jnp.float32),
            pltpu.VMEM((16,), jnp.float32),
            pltpu.SemaphoreType.DMA,
            pltpu.SemaphoreType.DMA,
        ],
        compiler_params=pltpu.CompilerParams(use_tc_tiling_on_sc=True),
    )
    def body(pred, target, out, pb0, tb0, pb1, tb1, accbuf, sem0, sem1):
        c = lax.axis_index("c")
        s = lax.axis_index("s")
        wid = s * 2 + c
        row0 = _TC_ROWS + wid * _ROWS_PER_W

        def start(buf_p, buf_t, sem, ci):
            row = row0 + (ci >> 2) * _CR
            col = (ci & 3) * _CC
            pltpu.async_copy(
                pred.at[pl.ds(row, _CR), pl.ds(col, _CC)], buf_p, sem
            )
            pltpu.async_copy(
                target.at[pl.ds(row, _CR), pl.ds(col, _CC)], buf_t, sem
            )

        def wait(buf_p, buf_t, sem):
            pltpu.make_async_copy(
                pred.at[pl.ds(0, _CR), pl.ds(0, _CC)], buf_p, sem
            ).wait()
            pltpu.make_async_copy(
                target.at[pl.ds(0, _CR), pl.ds(0, _CC)], buf_t, sem
            ).wait()

        def consume(buf_p, buf_t, accs):
            for r in range(_CR):

                @plsc.parallel_loop(
                    0, _VECS_PER_ROW, step=16, unroll=2, carry=accs
                )
                def inner(i, acc_t):
                    acc4 = list(acc_t)
                    for u in range(16):
                        idx = (i + u) * 16
                        p = buf_p[r, pl.ds(idx, 16)]
                        t = buf_t[r, pl.ds(idx, 16)]
                        d = jnp.minimum(jnp.maximum(p, 0.0), 1.0) - t
                        acc4[u % 4] = acc4[u % 4] + d * d
                    return tuple(acc4)

                accs = inner
            return accs

        start(pb0, tb0, sem0, 0)

        zero = jnp.zeros((16,), jnp.float32)

        def pair(g, accs):
            start(pb1, tb1, sem1, 2 * g + 1)
            wait(pb0, tb0, sem0)
            accs = consume(pb0, tb0, accs)

            @pl.when(g < _NCHUNKS // 2 - 1)
            def _():
                start(pb0, tb0, sem0, 2 * g + 2)

            wait(pb1, tb1, sem1)
            accs = consume(pb1, tb1, accs)
            return accs

        accs = lax.fori_loop(0, _NCHUNKS // 2, pair, (zero, zero, zero, zero))
        accbuf[...] = (accs[0] + accs[1]) + (accs[2] + accs[3])
        pltpu.sync_copy(accbuf, out.at[wid])

    return body(pred, target)


def _tc_body(p_ref, t_ref, o_ref, acc_ref):
    i = pl.program_id(0)

    @pl.when(i == 0)
    def _init():
        acc_ref[...] = jnp.zeros_like(acc_ref)

    d = jnp.clip(p_ref[...], 0.0, 1.0) - t_ref[...]
    acc_ref[...] += jnp.sum(d * d, axis=0, keepdims=True)

    @pl.when(i == _TC_GRID - 1)
    def _fin():
        o_ref[0, 0] = jnp.sum(acc_ref[...])


def _tc_partial(pred, target):
    return pl.pallas_call(
        _tc_body,
        grid=(_TC_GRID,),
        in_specs=[
            pl.BlockSpec((_BLOCK_ROWS, _N), lambda i: (i, 0)),
            pl.BlockSpec((_BLOCK_ROWS, _N), lambda i: (i, 0)),
        ],
        out_specs=pl.BlockSpec(memory_space=pltpu.SMEM),
        out_shape=jax.ShapeDtypeStruct((1, 1), jnp.float32),
        scratch_shapes=[pltpu.VMEM((1, _N), jnp.float32)],
        compiler_params=pltpu.CompilerParams(
            dimension_semantics=("arbitrary",),
        ),
    )(pred, target)


def kernel(pred, target):
    sc_out = _sc_partial(pred, target)
    tc_out = _tc_partial(pred, target)
    return (tc_out[0, 0] + jnp.sum(sc_out)) * (1.0 / float(_NUMEL))


# speedup vs baseline: 3.8576x; 1.1374x over previous
import jax
import jax.numpy as jnp
from jax.experimental import pallas as pl
from jax.experimental.pallas import tpu as pltpu

_N = 8192
_BLOCK_ROWS = 256
_GRID = _N // _BLOCK_ROWS
_NUMEL = float(_N) * float(_N)


def _body(p_ref, t_ref, o_ref, acc_ref):
    i = pl.program_id(0)

    @pl.when(i == 0)
    def _init():
        acc_ref[...] = jnp.zeros_like(acc_ref)

    d = jnp.clip(p_ref[...], 0.0, 1.0) - t_ref[...]
    acc_ref[...] += jnp.sum(d * d, axis=0, keepdims=True)

    @pl.when(i == _GRID - 1)
    def _fin():
        o_ref[0, 0] = jnp.sum(acc_ref[...]) * (1.0 / _NUMEL)


def kernel(pred, target):
    out = pl.pallas_call(
        _body,
        grid=(_GRID,),
        in_specs=[
            pl.BlockSpec((_BLOCK_ROWS, _N), lambda i: (i, 0)),
            pl.BlockSpec((_BLOCK_ROWS, _N), lambda i: (i, 0)),
        ],
        out_specs=pl.BlockSpec(memory_space=pltpu.SMEM),
        out_shape=jax.ShapeDtypeStruct((1, 1), jnp.float32),
        scratch_shapes=[pltpu.VMEM((1, _N), jnp.float32)],
        compiler_params=pltpu.CompilerParams(
            dimension_semantics=("arbitrary",),
        ),
    )(pred, target)
    return out[0, 0]
